# baseline (speedup 1.0000x reference)
import jax
import jax.numpy as jnp
from jax.experimental import pallas as pl
from jax.experimental.pallas import tpu as pltpu

N = 50000
D = 512
NG = 500
GF = 16
STEPS = 6
DL = 64
NT = 12

B = 10000
NB = N // B
W = 128
NGP = 512
NEG = -1e30


def _kernel(x_ref, batch_ref, Wa_ref, Wb_ref, bias_ref,
            attr_ref, W1_ref, b1_ref, gamma_ref, beta_ref, W2_ref, b2_ref,
            out_ref,
            h_ref, c_ref, m_ref, s_ref, r_ref):
    step = pl.program_id(0)
    blk = pl.program_id(1)

    def dot_t(a, b):
        return jax.lax.dot_general(a, b, (((1,), (1,)), ((), ())),
                                   preferred_element_type=jnp.float32)

    def finalize_r():
        return r_ref[...] * (1.0 / (s_ref[...] + 1e-16))

    @pl.when(blk == 0)
    def _step_start():
        @pl.when(step == 0)
        def _init():
            h_ref[...] = jnp.zeros((NGP, D), jnp.float32)
            c_ref[...] = jnp.zeros((NGP, D), jnp.float32)
            s_ref[...] = jnp.zeros((NGP, 1), jnp.float32)
            r_ref[...] = jnp.zeros((NGP, D), jnp.float32)

        rf = finalize_r()
        gates = dot_t(h_ref[...], Wa_ref[...]) + dot_t(rf, Wb_ref[...]) \
            + bias_ref[...]
        i = jax.nn.sigmoid(gates[:, 0:D])
        f = jax.nn.sigmoid(gates[:, D:2 * D])
        g = jnp.tanh(gates[:, 2 * D:3 * D])
        o = jax.nn.sigmoid(gates[:, 3 * D:4 * D])
        c_new = f * c_ref[...] + i * g
        h_ref[...] = o * jnp.tanh(c_new)
        c_ref[...] = c_new
        m_ref[...] = jnp.full((NGP, 1), NEG, jnp.float32)
        s_ref[...] = jnp.zeros((NGP, 1), jnp.float32)
        r_ref[...] = jnp.zeros((NGP, D), jnp.float32)

    x = x_ref[...]
    bb = batch_ref[0]
    gmin = jnp.min(bb)
    gmax = jnp.max(bb)
    g8 = (gmin // 8) * 8
    nchunk = (gmax - g8) // W + 1

    def chunk_body(k, _):
        cstart = g8 + k * W
        lo = jnp.minimum(cstart, NGP - W)
        q_win = h_ref[pl.ds(lo, W), :].astype(jnp.bfloat16)
        st = dot_t(q_win, x)
        rows = jax.lax.broadcasted_iota(jnp.int32, (W, B), 0) + lo
        in_chunk = (bb >= cstart) & (bb < cstart + W)
        maskT = (rows == bb) & in_chunk
        masked_st = jnp.where(maskT, st, NEG)
        bm = jnp.max(masked_st, axis=1, keepdims=True)
        m_old = m_ref[pl.ds(lo, W), :]
        m_new = jnp.maximum(m_old, bm)
        m_ref[pl.ds(lo, W), :] = m_new
        alpha = jnp.exp(m_old - m_new)
        wm = jnp.exp(masked_st - jnp.maximum(m_new, -1e29))
        s_old = s_ref[pl.ds(lo, W), :]
        s_ref[pl.ds(lo, W), :] = alpha * s_old + jnp.sum(wm, axis=1,
                                                         keepdims=True)
        blk_r = jax.lax.dot_general(wm.astype(jnp.bfloat16), x,
                                    (((1,), (0,)), ((), ())),
                                    preferred_element_type=jnp.float32)
        r_ref[pl.ds(lo, W), :] = alpha * r_ref[pl.ds(lo, W), :] + blk_r
        return 0

    jax.lax.fori_loop(0, nchunk, chunk_body, 0)

    @pl.when((step == STEPS - 1) & (blk == NB - 1))
    def _head():
        qst = jnp.concatenate([h_ref[...], finalize_r()], axis=1)
        out1 = (dot_t(qst, W1_ref[:, :2 * D])
                + dot_t(attr_ref[...], W1_ref[:, 2 * D:])
                + b1_ref[...])
        out1 = jnp.maximum(out1, 0.0)
        mu = jnp.mean(out1, axis=1, keepdims=True)
        dlt = out1 - mu
        var = jnp.mean(dlt * dlt, axis=1, keepdims=True)
        normed = dlt * jax.lax.rsqrt(var + 1e-5) * gamma_ref[...] + beta_ref[...]
        out_ref[...] = dot_t(normed, W2_ref[...]) + b2_ref[...]


@jax.jit
def kernel(node_features, batch, graph_attr, W_ih, W_hh, b_ih, b_hh,
           W1, b1, gamma, beta, W2, b2):
    x16 = node_features.astype(jnp.bfloat16)
    batch3 = batch.astype(jnp.int32).reshape(NB, 1, B)
    attr_p = jnp.zeros((NGP, GF), jnp.float32).at[:NG].set(graph_attr)
    W_a = W_ih[:, :D] + W_hh
    W_b = W_ih[:, D:]
    bias = (b_ih + b_hh).reshape(1, -1)

    grid = (STEPS, NB)
    whole = lambda shape: pl.BlockSpec(shape, lambda s, b: (0,) * len(shape))
    out = pl.pallas_call(
        _kernel,
        grid=grid,
        in_specs=[
            pl.BlockSpec((B, D), lambda s, b: (b, 0)),
            pl.BlockSpec((1, 1, B), lambda s, b: (b, 0, 0)),
            whole((4 * D, D)),
            whole((4 * D, D)),
            whole((1, 4 * D)),
            whole((NGP, GF)),
            whole((DL, 2 * D + GF)),
            whole((1, DL)),
            whole((1, DL)),
            whole((1, DL)),
            whole((NT, DL)),
            whole((1, NT)),
        ],
        out_specs=pl.BlockSpec((NGP, NT), lambda s, b: (0, 0)),
        out_shape=jax.ShapeDtypeStruct((NGP, NT), jnp.float32),
        scratch_shapes=[
            pltpu.VMEM((NGP, D), jnp.float32),
            pltpu.VMEM((NGP, D), jnp.float32),
            pltpu.VMEM((NGP, 1), jnp.float32),
            pltpu.VMEM((NGP, 1), jnp.float32),
            pltpu.VMEM((NGP, D), jnp.float32),
        ],
        compiler_params=pltpu.CompilerParams(
            dimension_semantics=("arbitrary", "arbitrary"),
        ),
    )(x16, batch3, W_a, W_b, bias, attr_p, W1,
      b1.reshape(1, -1), gamma.reshape(1, -1), beta.reshape(1, -1),
      W2, b2.reshape(1, -1))
    return out[:NG, :NT]


# speedup vs baseline: 1.1445x; 1.1445x over previous
import jax
import jax.numpy as jnp
from jax.experimental import pallas as pl
from jax.experimental.pallas import tpu as pltpu

N = 50000
D = 512
NG = 500
GF = 16
STEPS = 6
DL = 64
NT = 12

B = 5000
NB = N // B
W = 64
NGP = 512
NEG = -1e30


def _kernel(x_ref, batch_ref, Wa_ref, Wb_ref, bias_ref,
            attr_ref, W1_ref, b1_ref, gamma_ref, beta_ref, W2_ref, b2_ref,
            out_ref,
            h_ref, c_ref, m_ref, s_ref, r_ref):
    step = pl.program_id(0)
    blk = pl.program_id(1)

    def dot_t(a, b):
        return jax.lax.dot_general(a, b, (((1,), (1,)), ((), ())),
                                   preferred_element_type=jnp.float32)

    def finalize_r():
        return r_ref[...] * (1.0 / (s_ref[...] + 1e-16))

    @pl.when(blk == 0)
    def _step_start():
        @pl.when(step == 0)
        def _init():
            h_ref[...] = jnp.zeros((NGP, D), jnp.float32)
            c_ref[...] = jnp.zeros((NGP, D), jnp.float32)
            s_ref[...] = jnp.zeros((NGP, 1), jnp.float32)
            r_ref[...] = jnp.zeros((NGP, D), jnp.float32)

        rf = finalize_r()
        gates = dot_t(h_ref[...], Wa_ref[...]) + dot_t(rf, Wb_ref[...]) \
            + bias_ref[...]
        i = jax.nn.sigmoid(gates[:, 0:D])
        f = jax.nn.sigmoid(gates[:, D:2 * D])
        g = jnp.tanh(gates[:, 2 * D:3 * D])
        o = jax.nn.sigmoid(gates[:, 3 * D:4 * D])
        c_new = f * c_ref[...] + i * g
        h_ref[...] = o * jnp.tanh(c_new)
        c_ref[...] = c_new
        m_ref[...] = jnp.full((NGP, 1), NEG, jnp.float32)
        s_ref[...] = jnp.zeros((NGP, 1), jnp.float32)
        r_ref[...] = jnp.zeros((NGP, D), jnp.float32)

    x = x_ref[...].astype(jnp.bfloat16)
    bb = batch_ref[0]
    gmin = jnp.min(bb)
    gmax = jnp.max(bb)
    g8 = (gmin // 8) * 8
    nchunk = (gmax - g8) // W + 1

    def chunk_body(k, _):
        cstart = g8 + k * W
        lo = jnp.minimum(cstart, NGP - W)
        q_win = h_ref[pl.ds(lo, W), :].astype(jnp.bfloat16)
        st = dot_t(q_win, x)
        rows = jax.lax.broadcasted_iota(jnp.int32, (W, B), 0) + lo
        in_chunk = (bb >= cstart) & (bb < cstart + W)
        maskT = (rows == bb) & in_chunk
        masked_st = jnp.where(maskT, st, NEG)
        bm = jnp.max(masked_st, axis=1, keepdims=True)
        m_old = m_ref[pl.ds(lo, W), :]
        m_new = jnp.maximum(m_old, bm)
        m_ref[pl.ds(lo, W), :] = m_new
        alpha = jnp.exp(m_old - m_new)
        wm = jnp.exp(masked_st - jnp.maximum(m_new, -1e29))
        s_old = s_ref[pl.ds(lo, W), :]
        s_ref[pl.ds(lo, W), :] = alpha * s_old + jnp.sum(wm, axis=1,
                                                         keepdims=True)
        blk_r = jax.lax.dot_general(wm.astype(jnp.bfloat16), x,
                                    (((1,), (0,)), ((), ())),
                                    preferred_element_type=jnp.float32)
        r_ref[pl.ds(lo, W), :] = alpha * r_ref[pl.ds(lo, W), :] + blk_r
        return 0

    jax.lax.fori_loop(0, nchunk, chunk_body, 0)

    @pl.when((step == STEPS - 1) & (blk == NB - 1))
    def _head():
        qst = jnp.concatenate([h_ref[...], finalize_r()], axis=1)
        out1 = (dot_t(qst, W1_ref[:, :2 * D])
                + dot_t(attr_ref[...], W1_ref[:, 2 * D:])
                + b1_ref[...])
        out1 = jnp.maximum(out1, 0.0)
        mu = jnp.mean(out1, axis=1, keepdims=True)
        dlt = out1 - mu
        var = jnp.mean(dlt * dlt, axis=1, keepdims=True)
        normed = dlt * jax.lax.rsqrt(var + 1e-5) * gamma_ref[...] + beta_ref[...]
        out_ref[...] = dot_t(normed, W2_ref[...]) + b2_ref[...]


@jax.jit
def kernel(node_features, batch, graph_attr, W_ih, W_hh, b_ih, b_hh,
           W1, b1, gamma, beta, W2, b2):
    x16 = node_features
    batch3 = batch.astype(jnp.int32).reshape(NB, 1, B)
    attr_p = jnp.zeros((NGP, GF), jnp.float32).at[:NG].set(graph_attr)
    W_a = W_ih[:, :D] + W_hh
    W_b = W_ih[:, D:]
    bias = (b_ih + b_hh).reshape(1, -1)

    grid = (STEPS, NB)
    whole = lambda shape: pl.BlockSpec(shape, lambda s, b: (0,) * len(shape))
    out = pl.pallas_call(
        _kernel,
        grid=grid,
        in_specs=[
            pl.BlockSpec((B, D), lambda s, b: (b, 0)),
            pl.BlockSpec((1, 1, B), lambda s, b: (b, 0, 0)),
            whole((4 * D, D)),
            whole((4 * D, D)),
            whole((1, 4 * D)),
            whole((NGP, GF)),
            whole((DL, 2 * D + GF)),
            whole((1, DL)),
            whole((1, DL)),
            whole((1, DL)),
            whole((NT, DL)),
            whole((1, NT)),
        ],
        out_specs=pl.BlockSpec((NGP, NT), lambda s, b: (0, 0)),
        out_shape=jax.ShapeDtypeStruct((NGP, NT), jnp.float32),
        scratch_shapes=[
            pltpu.VMEM((NGP, D), jnp.float32),
            pltpu.VMEM((NGP, D), jnp.float32),
            pltpu.VMEM((NGP, 1), jnp.float32),
            pltpu.VMEM((NGP, 1), jnp.float32),
            pltpu.VMEM((NGP, D), jnp.float32),
        ],
        compiler_params=pltpu.CompilerParams(
            dimension_semantics=("arbitrary", "arbitrary"),
        ),
    )(x16, batch3, W_a, W_b, bias, attr_p, W1,
      b1.reshape(1, -1), gamma.reshape(1, -1), beta.reshape(1, -1),
      W2, b2.reshape(1, -1))
    return out[:NG, :NT]


# speedup vs baseline: 1.1555x; 1.0096x over previous
import jax
import jax.numpy as jnp
from jax.experimental import pallas as pl
from jax.experimental.pallas import tpu as pltpu

N = 50000
D = 512
NG = 500
GF = 16
STEPS = 6
DL = 64
NT = 12

B = 5000
NB = N // B
W = 64
NGP = 512
NEG = -1e30


def _dot_t(a, b):
    return jax.lax.dot_general(a, b, (((1,), (1,)), ((), ())),
                               preferred_element_type=jnp.float32)


def _attention_block(x, bb, h_ref, m_ref, s_ref, r_ref):
    gmin = jnp.min(bb)
    gmax = jnp.max(bb)
    g8 = (gmin // 8) * 8
    nchunk = (gmax - g8) // W + 1

    def chunk_body(k, _):
        cstart = g8 + k * W
        lo = jnp.minimum(cstart, NGP - W)
        q_win = h_ref[pl.ds(lo, W), :].astype(jnp.bfloat16)
        st = _dot_t(q_win, x)
        rows = jax.lax.broadcasted_iota(jnp.int32, (W, B), 0) + lo
        in_chunk = (bb >= cstart) & (bb < cstart + W)
        maskT = (rows == bb) & in_chunk
        masked_st = jnp.where(maskT, st, NEG)
        bm = jnp.max(masked_st, axis=1, keepdims=True)
        m_old = m_ref[pl.ds(lo, W), :]
        m_new = jnp.maximum(m_old, bm)
        m_ref[pl.ds(lo, W), :] = m_new
        alpha = jnp.exp(m_old - m_new)
        wm = jnp.exp(masked_st - jnp.maximum(m_new, -1e29))
        s_old = s_ref[pl.ds(lo, W), :]
        s_ref[pl.ds(lo, W), :] = alpha * s_old + jnp.sum(wm, axis=1,
                                                         keepdims=True)
        blk_r = jax.lax.dot_general(wm.astype(jnp.bfloat16), x,
                                    (((1,), (0,)), ((), ())),
                                    preferred_element_type=jnp.float32)
        r_ref[pl.ds(lo, W), :] = alpha * r_ref[pl.ds(lo, W), :] + blk_r
        return 0

    jax.lax.fori_loop(0, nchunk, chunk_body, 0)


def _kernel_a(x_ref, batch_ref, bias_ref,
              x16_ref, h_ref, c_ref, m_ref, s_ref, r_ref):
    blk = pl.program_id(0)

    @pl.when(blk == 0)
    def _init():
        bias = bias_ref[...]
        i = jax.nn.sigmoid(bias[:, 0:D])
        f = jax.nn.sigmoid(bias[:, D:2 * D])
        g = jnp.tanh(bias[:, 2 * D:3 * D])
        o = jax.nn.sigmoid(bias[:, 3 * D:4 * D])
        c0 = i * g
        h0 = o * jnp.tanh(c0)
        c_ref[...] = jnp.broadcast_to(c0, (NGP, D))
        h_ref[...] = jnp.broadcast_to(h0, (NGP, D))
        m_ref[...] = jnp.full((NGP, 1), NEG, jnp.float32)
        s_ref[...] = jnp.zeros((NGP, 1), jnp.float32)
        r_ref[...] = jnp.zeros((NGP, D), jnp.float32)

    x = x_ref[...].astype(jnp.bfloat16)
    x16_ref[...] = x
    _attention_block(x, batch_ref[0], h_ref, m_ref, s_ref, r_ref)


def _kernel_b(x16_ref, batch_ref, Wa_ref, Wb_ref, bias_ref,
              attr_ref, W1_ref, b1_ref, gamma_ref, beta_ref, W2_ref, b2_ref,
              h_in, c_in, m_in, s_in, r_in,
              out_ref,
              h_ref, c_ref, m_ref, s_ref, r_ref):
    step = pl.program_id(0)
    blk = pl.program_id(1)

    def finalize_r():
        return r_ref[...] * (1.0 / (s_ref[...] + 1e-16))

    @pl.when(blk == 0)
    def _step_start():
        @pl.when(step == 0)
        def _carry_in():
            h_ref[...] = h_in[...]
            c_ref[...] = c_in[...]
            m_ref[...] = m_in[...]
            s_ref[...] = s_in[...]
            r_ref[...] = r_in[...]

        rf = finalize_r()
        gates = _dot_t(h_ref[...], Wa_ref[...]) + _dot_t(rf, Wb_ref[...]) \
            + bias_ref[...]
        i = jax.nn.sigmoid(gates[:, 0:D])
        f = jax.nn.sigmoid(gates[:, D:2 * D])
        g = jnp.tanh(gates[:, 2 * D:3 * D])
        o = jax.nn.sigmoid(gates[:, 3 * D:4 * D])
        c_new = f * c_ref[...] + i * g
        h_ref[...] = o * jnp.tanh(c_new)
        c_ref[...] = c_new
        m_ref[...] = jnp.full((NGP, 1), NEG, jnp.float32)
        s_ref[...] = jnp.zeros((NGP, 1), jnp.float32)
        r_ref[...] = jnp.zeros((NGP, D), jnp.float32)

    _attention_block(x16_ref[...], batch_ref[0], h_ref, m_ref, s_ref, r_ref)

    @pl.when((step == STEPS - 2) & (blk == NB - 1))
    def _head():
        qst = jnp.concatenate([h_ref[...], finalize_r()], axis=1)
        out1 = (_dot_t(qst, W1_ref[:, :2 * D])
                + _dot_t(attr_ref[...], W1_ref[:, 2 * D:])
                + b1_ref[...])
        out1 = jnp.maximum(out1, 0.0)
        mu = jnp.mean(out1, axis=1, keepdims=True)
        dlt = out1 - mu
        var = jnp.mean(dlt * dlt, axis=1, keepdims=True)
        normed = dlt * jax.lax.rsqrt(var + 1e-5) * gamma_ref[...] + beta_ref[...]
        out_ref[...] = _dot_t(normed, W2_ref[...]) + b2_ref[...]


@jax.jit
def kernel(node_features, batch, graph_attr, W_ih, W_hh, b_ih, b_hh,
           W1, b1, gamma, beta, W2, b2):
    batch3 = batch.astype(jnp.int32).reshape(NB, 1, B)
    attr_p = jnp.zeros((NGP, GF), jnp.float32).at[:NG].set(graph_attr)
    W_a = W_ih[:, :D] + W_hh
    W_b = W_ih[:, D:]
    bias = (b_ih + b_hh).reshape(1, -1)

    wh = lambda *shape: pl.BlockSpec(shape, lambda *_: (0,) * len(shape))
    fdt = jnp.float32

    x16, h0, c0, m0, s0, r0 = pl.pallas_call(
        _kernel_a,
        grid=(NB,),
        in_specs=[
            pl.BlockSpec((B, D), lambda b: (b, 0)),
            pl.BlockSpec((1, 1, B), lambda b: (b, 0, 0)),
            wh(1, 4 * D),
        ],
        out_specs=[
            pl.BlockSpec((B, D), lambda b: (b, 0)),
            wh(NGP, D), wh(NGP, D), wh(NGP, 1), wh(NGP, 1), wh(NGP, D),
        ],
        out_shape=[
            jax.ShapeDtypeStruct((N, D), jnp.bfloat16),
            jax.ShapeDtypeStruct((NGP, D), fdt),
            jax.ShapeDtypeStruct((NGP, D), fdt),
            jax.ShapeDtypeStruct((NGP, 1), fdt),
            jax.ShapeDtypeStruct((NGP, 1), fdt),
            jax.ShapeDtypeStruct((NGP, D), fdt),
        ],
        compiler_params=pltpu.CompilerParams(
            dimension_semantics=("arbitrary",),
        ),
    )(node_features, batch3, bias)

    out = pl.pallas_call(
        _kernel_b,
        grid=(STEPS - 1, NB),
        in_specs=[
            pl.BlockSpec((B, D), lambda s, b: (b, 0)),
            pl.BlockSpec((1, 1, B), lambda s, b: (b, 0, 0)),
            wh(4 * D, D),
            wh(4 * D, D),
            wh(1, 4 * D),
            wh(NGP, GF),
            wh(DL, 2 * D + GF),
            wh(1, DL),
            wh(1, DL),
            wh(1, DL),
            wh(NT, DL),
            wh(1, NT),
            wh(NGP, D), wh(NGP, D), wh(NGP, 1), wh(NGP, 1), wh(NGP, D),
        ],
        out_specs=pl.BlockSpec((NGP, NT), lambda s, b: (0, 0)),
        out_shape=jax.ShapeDtypeStruct((NGP, NT), jnp.float32),
        scratch_shapes=[
            pltpu.VMEM((NGP, D), fdt),
            pltpu.VMEM((NGP, D), fdt),
            pltpu.VMEM((NGP, 1), fdt),
            pltpu.VMEM((NGP, 1), fdt),
            pltpu.VMEM((NGP, D), fdt),
        ],
        compiler_params=pltpu.CompilerParams(
            dimension_semantics=("arbitrary", "arbitrary"),
        ),
    )(x16, batch3, W_a, W_b, bias, attr_p, W1,
      b1.reshape(1, -1), gamma.reshape(1, -1), beta.reshape(1, -1),
      W2, b2.reshape(1, -1), h0, c0, m0, s0, r0)
    return out[:NG, :NT]
